# baseline (speedup 1.0000x reference)
import jax
import jax.numpy as jnp
from jax.experimental import pallas as pl
from jax.experimental.pallas import tpu as pltpu
from jax.experimental.pallas import tpu_sc as plsc

_NUM_E = 100000
_NUM_R = 100000
_D = 32
_B = 1024
_MARGIN = 1.0
_WEIGHT_SOFT = 0.01
_ORTH_C = 100000 * 0.05

_BLK = 5000
_NSTREAM = 4
_G = _NUM_E // (_BLK * _NSTREAM)


def _sc_gather(idx_e, idx_r, table):
    mesh = plsc.VectorSubcoreMesh(core_axis_name="c", subcore_axis_name="s")
    out_type = (
        jax.ShapeDtypeStruct((4 * _B, 128), jnp.float32),
        jax.ShapeDtypeStruct((2 * _B, 128), jnp.float32),
    )

    @pl.kernel(out_type=out_type, mesh=mesh)
    def gather_kernel(ie_hbm, ir_hbm, t_hbm, ge_hbm, gr_hbm):
        def body_e(i_vmem, o_vmem):
            pltpu.sync_copy(t_hbm.at[i_vmem.at[0]], o_vmem)

        pltpu.emit_pipeline(
            body_e,
            grid=(4 * _B // 128,),
            in_specs=[pl.BlockSpec((1, 128), lambda i: (0, i))],
            out_specs=[pl.BlockSpec((128, 128), lambda i: (i, 0))],
            core_axis_name=("c", "s"),
            dimension_semantics=(pltpu.PARALLEL,),
        )(ie_hbm, ge_hbm)

        def body_r(i_vmem, o_vmem):
            pltpu.sync_copy(t_hbm.at[i_vmem.at[0]], o_vmem)

        pltpu.emit_pipeline(
            body_r,
            grid=(2 * _B // 128,),
            in_specs=[pl.BlockSpec((1, 128), lambda i: (0, i))],
            out_specs=[pl.BlockSpec((128, 128), lambda i: (i, 0))],
            core_axis_name=("c", "s"),
            dimension_semantics=(pltpu.PARALLEL,),
        )(ir_hbm, gr_hbm)

    return gather_kernel(idx_e, idx_r, table)


def _scan_body(*refs):
    t_refs = refs[:_NSTREAM]
    out_ref, acc_ref = refs[_NSTREAM], refs[_NSTREAM + 1]
    i = pl.program_id(0)

    @pl.when(i == 0)
    def _():
        acc_ref[0] = 0.0
        acc_ref[1] = 0.0

    ent = 0.0
    orth = 0.0
    for t_ref in t_refs:
        t = t_ref[...]
        e = t[:, 0:_D]
        n = t[:, 2 * _D:3 * _D]
        p = t[:, 3 * _D:4 * _D]

        ent += jnp.sum(jnp.abs(jnp.sum(e * e, axis=1) - float(_NUM_E)))

        ndp = jnp.sum(n * p, axis=1)
        nn = jnp.sum(n * n, axis=1)
        pp = jnp.sum(p * p, axis=1)
        orth += jnp.sum(jnp.abs((ndp * ndp) / (nn * pp) - _ORTH_C))
    acc_ref[0] += ent
    acc_ref[1] += orth

    @pl.when(i == _G - 1)
    def _():
        out_ref[0, 0] = acc_ref[0]
        out_ref[0, 1] = acc_ref[1]


def _scan_call(table):
    specs = [
        pl.BlockSpec((_BLK, 128), lambda i, k=k: (k * _G + i, 0))
        for k in range(_NSTREAM)
    ]
    return pl.pallas_call(
        _scan_body,
        grid=(_G,),
        in_specs=specs,
        out_specs=pl.BlockSpec(memory_space=pltpu.SMEM),
        out_shape=jax.ShapeDtypeStruct((1, 2), jnp.float32),
        scratch_shapes=[pltpu.SMEM((2,), jnp.float32)],
    )(*([table] * _NSTREAM))


def _batch_body(ge_ref, grn_ref, c_ref, out_ref):
    ge = ge_ref[:, 0:_D]
    gr = grn_ref[:, _D:2 * _D]
    gn = grn_ref[:, 2 * _D:3 * _D]
    nv = gn * jax.lax.rsqrt(jnp.sum(gn * gn, axis=1, keepdims=True))

    def scores(h, t, r, v):
        hd = h - t
        a = hd + r
        b = v * hd
        s1 = jnp.sum(b, axis=0, keepdims=True)
        s2 = jnp.sum(b * b, axis=0, keepdims=True)
        q = float(_B) * a * a - 2.0 * a * s1 + s2
        return jnp.sqrt(jnp.maximum(q, 0.0))

    sp = scores(ge[0:_B], ge[_B:2 * _B], gr[0:_B], nv[0:_B])
    sn = scores(ge[2 * _B:3 * _B], ge[3 * _B:4 * _B], gr[_B:2 * _B],
                nv[_B:2 * _B])
    margin = jnp.sum(jnp.maximum(0.0, sp - sn + _MARGIN))
    out_ref[0, 0] = margin + _WEIGHT_SOFT * (c_ref[0, 0] + c_ref[0, 1])


def _batch_call(ge, grn, consts):
    return pl.pallas_call(
        _batch_body,
        grid=(1,),
        in_specs=[
            pl.BlockSpec((4 * _B, 128), lambda i: (0, 0)),
            pl.BlockSpec((2 * _B, 128), lambda i: (0, 0)),
            pl.BlockSpec(memory_space=pltpu.SMEM),
        ],
        out_specs=pl.BlockSpec(memory_space=pltpu.SMEM),
        out_shape=jax.ShapeDtypeStruct((1, 1), jnp.float32),
    )(ge, grn, consts)


def kernel(batch_positives, batch_negatives, entity_emb, relation_emb,
           projected_relation_emb, normal_vector_emb):
    idx_e = jnp.concatenate([
        batch_positives[:, 0], batch_positives[:, 2],
        batch_negatives[:, 0], batch_negatives[:, 2],
    ]).reshape(1, 4 * _B)
    idx_r = jnp.concatenate([
        batch_positives[:, 1], batch_negatives[:, 1],
    ]).reshape(1, 2 * _B)

    table = jnp.concatenate(
        [entity_emb, relation_emb, normal_vector_emb, projected_relation_emb],
        axis=1)

    ge, grn = _sc_gather(idx_e, idx_r, table)
    consts = _scan_call(table)
    out = _batch_call(ge, grn, consts)
    return out[0, 0]


# speedup vs baseline: 1.0943x; 1.0943x over previous
import jax
import jax.numpy as jnp
from jax.experimental import pallas as pl
from jax.experimental.pallas import tpu as pltpu
from jax.experimental.pallas import tpu_sc as plsc

_NUM_E = 100000
_NUM_R = 100000
_D = 32
_B = 1024
_MARGIN = 1.0
_WEIGHT_SOFT = 0.01
_ORTH_C = 100000 * 0.05

_BLK = 5000
_G = _NUM_E // (2 * _BLK)


def _sc_gather(idx_e, idx_r, ep, rp, np_):
    mesh = plsc.VectorSubcoreMesh(core_axis_name="c", subcore_axis_name="s")
    out_type = (
        jax.ShapeDtypeStruct((4 * _B, 128), jnp.float32),
        jax.ShapeDtypeStruct((2 * _B, 128), jnp.float32),
        jax.ShapeDtypeStruct((2 * _B, 128), jnp.float32),
    )

    @pl.kernel(out_type=out_type, mesh=mesh)
    def gather_kernel(ie_hbm, ir_hbm, e_hbm, r_hbm, n_hbm, ge_hbm, gr_hbm, gn_hbm):
        def body_e(i_vmem, o_vmem):
            pltpu.sync_copy(e_hbm.at[i_vmem.at[0]], o_vmem)

        pltpu.emit_pipeline(
            body_e,
            grid=(4 * _B // 128,),
            in_specs=[pl.BlockSpec((1, 128), lambda i: (0, i))],
            out_specs=[pl.BlockSpec((128, 128), lambda i: (i, 0))],
            core_axis_name=("c", "s"),
            dimension_semantics=(pltpu.PARALLEL,),
        )(ie_hbm, ge_hbm)

        def body_rn(i_vmem, or_vmem, on_vmem):
            pltpu.sync_copy(r_hbm.at[i_vmem.at[0]], or_vmem)
            pltpu.sync_copy(n_hbm.at[i_vmem.at[0]], on_vmem)

        pltpu.emit_pipeline(
            body_rn,
            grid=(2 * _B // 128,),
            in_specs=[pl.BlockSpec((1, 128), lambda i: (0, i))],
            out_specs=[
                pl.BlockSpec((128, 128), lambda i: (i, 0)),
                pl.BlockSpec((128, 128), lambda i: (i, 0)),
            ],
            core_axis_name=("c", "s"),
            dimension_semantics=(pltpu.PARALLEL,),
        )(ir_hbm, gr_hbm, gn_hbm)

    return gather_kernel(idx_e, idx_r, ep, rp, np_)


def _scan_body(e0_ref, e1_ref, n0_ref, n1_ref, p0_ref, p1_ref, out_ref, acc_ref):
    i = pl.program_id(0)

    @pl.when(i == 0)
    def _():
        acc_ref[0] = 0.0
        acc_ref[1] = 0.0

    ent = 0.0
    orth = 0.0
    for e_ref in (e0_ref, e1_ref):
        e = e_ref[:, 0:_D]
        ent += jnp.sum(jnp.abs(jnp.sum(e * e, axis=1) - float(_NUM_E)))
    for n_ref, p_ref in ((n0_ref, p0_ref), (n1_ref, p1_ref)):
        n = n_ref[:, 0:_D]
        p = p_ref[:, 0:_D]
        ndp = jnp.sum(n * p, axis=1)
        nn = jnp.sum(n * n, axis=1)
        pp = jnp.sum(p * p, axis=1)
        orth += jnp.sum(jnp.abs((ndp * ndp) / (nn * pp) - _ORTH_C))
    acc_ref[0] += ent
    acc_ref[1] += orth

    @pl.when(i == _G - 1)
    def _():
        out_ref[0, 0] = acc_ref[0]
        out_ref[0, 1] = acc_ref[1]


def _scan_call(ep, np_, pp_):
    lo = pl.BlockSpec((_BLK, 128), lambda i: (i, 0))
    hi = pl.BlockSpec((_BLK, 128), lambda i: (_G + i, 0))
    return pl.pallas_call(
        _scan_body,
        grid=(_G,),
        in_specs=[lo, hi, lo, hi, lo, hi],
        out_specs=pl.BlockSpec(memory_space=pltpu.SMEM),
        out_shape=jax.ShapeDtypeStruct((1, 2), jnp.float32),
        scratch_shapes=[pltpu.SMEM((2,), jnp.float32)],
    )(ep, ep, np_, np_, pp_, pp_)


def _batch_body(ge_ref, gr_ref, gn_ref, c_ref, out_ref):
    ge = ge_ref[:, 0:_D].astype(jnp.float32)
    gr = gr_ref[:, 0:_D].astype(jnp.float32)
    gn = gn_ref[:, 0:_D].astype(jnp.float32)
    nv = gn * jax.lax.rsqrt(jnp.sum(gn * gn, axis=1, keepdims=True))

    def scores(h, t, r, v):
        hd = h - t
        a = hd + r
        b = v * hd
        s1 = jnp.sum(b, axis=0, keepdims=True)
        s2 = jnp.sum(b * b, axis=0, keepdims=True)
        q = float(_B) * a * a - 2.0 * a * s1 + s2
        return jnp.sqrt(jnp.maximum(q, 0.0))

    sp = scores(ge[0:_B], ge[_B:2 * _B], gr[0:_B], nv[0:_B])
    sn = scores(ge[2 * _B:3 * _B], ge[3 * _B:4 * _B], gr[_B:2 * _B],
                nv[_B:2 * _B])
    margin = jnp.sum(jnp.maximum(0.0, sp - sn + _MARGIN))
    out_ref[0, 0] = margin + _WEIGHT_SOFT * (c_ref[0, 0] + c_ref[0, 1])


def _batch_call(ge, gr, gn, consts):
    return pl.pallas_call(
        _batch_body,
        grid=(1,),
        in_specs=[
            pl.BlockSpec((4 * _B, 128), lambda i: (0, 0)),
            pl.BlockSpec((2 * _B, 128), lambda i: (0, 0)),
            pl.BlockSpec((2 * _B, 128), lambda i: (0, 0)),
            pl.BlockSpec(memory_space=pltpu.SMEM),
        ],
        out_specs=pl.BlockSpec(memory_space=pltpu.SMEM),
        out_shape=jax.ShapeDtypeStruct((1, 1), jnp.float32),
    )(ge, gr, gn, consts)


def kernel(batch_positives, batch_negatives, entity_emb, relation_emb,
           projected_relation_emb, normal_vector_emb):
    idx_e = jnp.concatenate([
        batch_positives[:, 0], batch_positives[:, 2],
        batch_negatives[:, 0], batch_negatives[:, 2],
    ]).reshape(1, 4 * _B)
    idx_r = jnp.concatenate([
        batch_positives[:, 1], batch_negatives[:, 1],
    ]).reshape(1, 2 * _B)

    def prep(x):
        return jnp.pad(x, ((0, 0), (0, 128 - _D)))

    ep = prep(entity_emb)
    rp = prep(relation_emb)
    np_ = prep(normal_vector_emb)
    pp_ = prep(projected_relation_emb)

    ge, gr, gn = _sc_gather(idx_e, idx_r, ep, rp, np_)
    consts = _scan_call(ep, np_, pp_)
    out = _batch_call(ge, gr, gn, consts)
    return out[0, 0]


# speedup vs baseline: 1.0969x; 1.0023x over previous
import jax
import jax.numpy as jnp
from jax.experimental import pallas as pl
from jax.experimental.pallas import tpu as pltpu
from jax.experimental.pallas import tpu_sc as plsc

_NUM_E = 100000
_NUM_R = 100000
_D = 32
_B = 1024
_MARGIN = 1.0
_WEIGHT_SOFT = 0.01
_ORTH_C = 100000 * 0.05

_BLK = 10000
_G = _NUM_E // _BLK


def _sc_gather(idx_e, idx_r, ep, rp, np_):
    mesh = plsc.VectorSubcoreMesh(core_axis_name="c", subcore_axis_name="s")
    out_type = (
        jax.ShapeDtypeStruct((4 * _B, 128), jnp.float32),
        jax.ShapeDtypeStruct((2 * _B, 128), jnp.float32),
        jax.ShapeDtypeStruct((2 * _B, 128), jnp.float32),
    )

    @pl.kernel(out_type=out_type, mesh=mesh)
    def gather_kernel(ie_hbm, ir_hbm, e_hbm, r_hbm, n_hbm, ge_hbm, gr_hbm, gn_hbm):
        def body_e(i_vmem, o_vmem):
            pltpu.sync_copy(e_hbm.at[i_vmem.at[0]], o_vmem)

        pltpu.emit_pipeline(
            body_e,
            grid=(4 * _B // 128,),
            in_specs=[pl.BlockSpec((1, 128), lambda i: (0, i))],
            out_specs=[pl.BlockSpec((128, 128), lambda i: (i, 0))],
            core_axis_name=("c", "s"),
            dimension_semantics=(pltpu.PARALLEL,),
        )(ie_hbm, ge_hbm)

        def body_rn(i_vmem, or_vmem, on_vmem):
            pltpu.sync_copy(r_hbm.at[i_vmem.at[0]], or_vmem)
            pltpu.sync_copy(n_hbm.at[i_vmem.at[0]], on_vmem)

        pltpu.emit_pipeline(
            body_rn,
            grid=(2 * _B // 128,),
            in_specs=[pl.BlockSpec((1, 128), lambda i: (0, i))],
            out_specs=[
                pl.BlockSpec((128, 128), lambda i: (i, 0)),
                pl.BlockSpec((128, 128), lambda i: (i, 0)),
            ],
            core_axis_name=("c", "s"),
            dimension_semantics=(pltpu.PARALLEL,),
        )(ir_hbm, gr_hbm, gn_hbm)

    return gather_kernel(idx_e, idx_r, ep, rp, np_)


def _scan_body(e_ref, n_ref, p_ref, out_ref, acc_ref):
    i = pl.program_id(0)

    @pl.when(i == 0)
    def _():
        acc_ref[0] = 0.0
        acc_ref[1] = 0.0

    e = e_ref[:, 0:_D]
    acc_ref[0] += jnp.sum(jnp.abs(jnp.sum(e * e, axis=1) - float(_NUM_E)))

    n = n_ref[:, 0:_D]
    p = p_ref[:, 0:_D]
    ndp = jnp.sum(n * p, axis=1)
    nn = jnp.sum(n * n, axis=1)
    pp = jnp.sum(p * p, axis=1)
    acc_ref[1] += jnp.sum(jnp.abs((ndp * ndp) / (nn * pp) - _ORTH_C))

    @pl.when(i == _G - 1)
    def _():
        out_ref[0, 0] = acc_ref[0]
        out_ref[0, 1] = acc_ref[1]


def _scan_call(ep, np_, pp_):
    return pl.pallas_call(
        _scan_body,
        grid=(_G,),
        in_specs=[
            pl.BlockSpec((_BLK, 128), lambda i: (i, 0)),
            pl.BlockSpec((_BLK, 128), lambda i: (i, 0)),
            pl.BlockSpec((_BLK, 128), lambda i: (i, 0)),
        ],
        out_specs=pl.BlockSpec(memory_space=pltpu.SMEM),
        out_shape=jax.ShapeDtypeStruct((1, 2), jnp.float32),
        scratch_shapes=[pltpu.SMEM((2,), jnp.float32)],
    )(ep, np_, pp_)


def _batch_body(ge_ref, gr_ref, gn_ref, c_ref, out_ref):
    ge = ge_ref[:, 0:_D].astype(jnp.float32)
    gr = gr_ref[:, 0:_D].astype(jnp.float32)
    gn = gn_ref[:, 0:_D].astype(jnp.float32)
    nv = gn * jax.lax.rsqrt(jnp.sum(gn * gn, axis=1, keepdims=True))

    def scores(h, t, r, v):
        hd = h - t
        a = hd + r
        b = v * hd
        s1 = jnp.sum(b, axis=0, keepdims=True)
        s2 = jnp.sum(b * b, axis=0, keepdims=True)
        q = float(_B) * a * a - 2.0 * a * s1 + s2
        return jnp.sqrt(jnp.maximum(q, 0.0))

    sp = scores(ge[0:_B], ge[_B:2 * _B], gr[0:_B], nv[0:_B])
    sn = scores(ge[2 * _B:3 * _B], ge[3 * _B:4 * _B], gr[_B:2 * _B],
                nv[_B:2 * _B])
    margin = jnp.sum(jnp.maximum(0.0, sp - sn + _MARGIN))
    out_ref[0, 0] = margin + _WEIGHT_SOFT * (c_ref[0, 0] + c_ref[0, 1])


def _batch_call(ge, gr, gn, consts):
    return pl.pallas_call(
        _batch_body,
        grid=(1,),
        in_specs=[
            pl.BlockSpec((4 * _B, 128), lambda i: (0, 0)),
            pl.BlockSpec((2 * _B, 128), lambda i: (0, 0)),
            pl.BlockSpec((2 * _B, 128), lambda i: (0, 0)),
            pl.BlockSpec(memory_space=pltpu.SMEM),
        ],
        out_specs=pl.BlockSpec(memory_space=pltpu.SMEM),
        out_shape=jax.ShapeDtypeStruct((1, 1), jnp.float32),
    )(ge, gr, gn, consts)


def kernel(batch_positives, batch_negatives, entity_emb, relation_emb,
           projected_relation_emb, normal_vector_emb):
    idx_e = jnp.concatenate([
        batch_positives[:, 0], batch_positives[:, 2],
        batch_negatives[:, 0], batch_negatives[:, 2],
    ]).reshape(1, 4 * _B)
    idx_r = jnp.concatenate([
        batch_positives[:, 1], batch_negatives[:, 1],
    ]).reshape(1, 2 * _B)

    def prep(x):
        return jnp.pad(x, ((0, 0), (0, 128 - _D)))

    ep = prep(entity_emb)
    rp = prep(relation_emb)
    np_ = prep(normal_vector_emb)
    pp_ = prep(projected_relation_emb)

    ge, gr, gn = _sc_gather(idx_e, idx_r, ep, rp, np_)
    consts = _scan_call(ep, np_, pp_)
    out = _batch_call(ge, gr, gn, consts)
    return out[0, 0]


# speedup vs baseline: 1.1502x; 1.0486x over previous
import jax
import jax.numpy as jnp
from jax.experimental import pallas as pl
from jax.experimental.pallas import tpu as pltpu
from jax.experimental.pallas import tpu_sc as plsc

_NUM_E = 100000
_NUM_R = 100000
_D = 32
_B = 1024
_MARGIN = 1.0
_WEIGHT_SOFT = 0.01
_ORTH_C = 100000 * 0.05

_BLK = 10000
_G = _NUM_E // _BLK


def _sc_gather(idx_e, idx_r, ep, rp, np_):
    mesh = plsc.VectorSubcoreMesh(core_axis_name="c", subcore_axis_name="s")
    out_type = (
        jax.ShapeDtypeStruct((4 * _B, 128), jnp.float32),
        jax.ShapeDtypeStruct((2 * _B, 128), jnp.float32),
        jax.ShapeDtypeStruct((2 * _B, 128), jnp.float32),
    )

    @pl.kernel(out_type=out_type, mesh=mesh)
    def gather_kernel(ie_hbm, ir_hbm, e_hbm, r_hbm, n_hbm, ge_hbm, gr_hbm, gn_hbm):
        def body_e(i_vmem, o_vmem):
            pltpu.sync_copy(e_hbm.at[i_vmem.at[0]], o_vmem)

        pltpu.emit_pipeline(
            body_e,
            grid=(4 * _B // 128,),
            in_specs=[pl.BlockSpec((1, 128), lambda i: (0, i))],
            out_specs=[pl.BlockSpec((128, 128), lambda i: (i, 0))],
            core_axis_name=("c", "s"),
            dimension_semantics=(pltpu.PARALLEL,),
        )(ie_hbm, ge_hbm)

        def body_rn(i_vmem, or_vmem, on_vmem):
            pltpu.sync_copy(r_hbm.at[i_vmem.at[0]], or_vmem)
            pltpu.sync_copy(n_hbm.at[i_vmem.at[0]], on_vmem)

        pltpu.emit_pipeline(
            body_rn,
            grid=(2 * _B // 128,),
            in_specs=[pl.BlockSpec((1, 128), lambda i: (0, i))],
            out_specs=[
                pl.BlockSpec((128, 128), lambda i: (i, 0)),
                pl.BlockSpec((128, 128), lambda i: (i, 0)),
            ],
            core_axis_name=("c", "s"),
            dimension_semantics=(pltpu.PARALLEL,),
        )(ir_hbm, gr_hbm, gn_hbm)

    return gather_kernel(idx_e, idx_r, ep, rp, np_)


def _scan_body(e_ref, n_ref, p_ref, out_ref, acc_ref):
    i = pl.program_id(0)

    @pl.when(i == 0)
    def _():
        acc_ref[0] = 0.0
        acc_ref[1] = 0.0

    e = e_ref[:, 0:_D]
    acc_ref[0] += jnp.sum(jnp.abs(jnp.sum(e * e, axis=1) - float(_NUM_E)))

    n = n_ref[:, 0:_D]
    p = p_ref[...]
    ndp = jnp.sum(n * p, axis=1)
    nn = jnp.sum(n * n, axis=1)
    pp = jnp.sum(p * p, axis=1)
    acc_ref[1] += jnp.sum(jnp.abs((ndp * ndp) / (nn * pp) - _ORTH_C))

    @pl.when(i == _G - 1)
    def _():
        out_ref[0, 0] = acc_ref[0]
        out_ref[0, 1] = acc_ref[1]


def _scan_call(ep, np_, pp_):
    return pl.pallas_call(
        _scan_body,
        grid=(_G,),
        in_specs=[
            pl.BlockSpec((_BLK, 128), lambda i: (i, 0)),
            pl.BlockSpec((_BLK, 128), lambda i: (i, 0)),
            pl.BlockSpec((_BLK, _D), lambda i: (i, 0)),
        ],
        out_specs=pl.BlockSpec(memory_space=pltpu.SMEM),
        out_shape=jax.ShapeDtypeStruct((1, 2), jnp.float32),
        scratch_shapes=[pltpu.SMEM((2,), jnp.float32)],
    )(ep, np_, pp_)


def _batch_body(ge_ref, gr_ref, gn_ref, c_ref, out_ref):
    ge = ge_ref[:, 0:_D].astype(jnp.float32)
    gr = gr_ref[:, 0:_D].astype(jnp.float32)
    gn = gn_ref[:, 0:_D].astype(jnp.float32)
    nv = gn * jax.lax.rsqrt(jnp.sum(gn * gn, axis=1, keepdims=True))

    def scores(h, t, r, v):
        hd = h - t
        a = hd + r
        b = v * hd
        s1 = jnp.sum(b, axis=0, keepdims=True)
        s2 = jnp.sum(b * b, axis=0, keepdims=True)
        q = float(_B) * a * a - 2.0 * a * s1 + s2
        return jnp.sqrt(jnp.maximum(q, 0.0))

    sp = scores(ge[0:_B], ge[_B:2 * _B], gr[0:_B], nv[0:_B])
    sn = scores(ge[2 * _B:3 * _B], ge[3 * _B:4 * _B], gr[_B:2 * _B],
                nv[_B:2 * _B])
    margin = jnp.sum(jnp.maximum(0.0, sp - sn + _MARGIN))
    out_ref[0, 0] = margin + _WEIGHT_SOFT * (c_ref[0, 0] + c_ref[0, 1])


def _batch_call(ge, gr, gn, consts):
    return pl.pallas_call(
        _batch_body,
        grid=(1,),
        in_specs=[
            pl.BlockSpec((4 * _B, 128), lambda i: (0, 0)),
            pl.BlockSpec((2 * _B, 128), lambda i: (0, 0)),
            pl.BlockSpec((2 * _B, 128), lambda i: (0, 0)),
            pl.BlockSpec(memory_space=pltpu.SMEM),
        ],
        out_specs=pl.BlockSpec(memory_space=pltpu.SMEM),
        out_shape=jax.ShapeDtypeStruct((1, 1), jnp.float32),
    )(ge, gr, gn, consts)


def kernel(batch_positives, batch_negatives, entity_emb, relation_emb,
           projected_relation_emb, normal_vector_emb):
    idx_e = jnp.concatenate([
        batch_positives[:, 0], batch_positives[:, 2],
        batch_negatives[:, 0], batch_negatives[:, 2],
    ]).reshape(1, 4 * _B)
    idx_r = jnp.concatenate([
        batch_positives[:, 1], batch_negatives[:, 1],
    ]).reshape(1, 2 * _B)

    def prep(x):
        return jnp.pad(x, ((0, 0), (0, 128 - _D)))

    ep = prep(entity_emb)
    rp = prep(relation_emb)
    np_ = prep(normal_vector_emb)
    pp_ = projected_relation_emb

    ge, gr, gn = _sc_gather(idx_e, idx_r, ep, rp, np_)
    consts = _scan_call(ep, np_, pp_)
    out = _batch_call(ge, gr, gn, consts)
    return out[0, 0]
